# baseline (speedup 1.0000x reference)
import functools

import jax
import numpy as np
import jax.numpy as jnp
from jax import lax
from jax.experimental import pallas as pl
from jax.experimental.pallas import tpu as pltpu
from jax.experimental.pallas import tpu_sc as plsc

ROWS, COLS = 64, 32768
N = ROWS * COLS
RANK = N // 2 - 1
L = 16
NC, NS = 2, 16
W = NC * NS
RPW = ROWS // W
CHUNK = N // W
VECS = CHUNK // L
SHIFTS = (21, 10, 0)
NBS = (2048, 2048, 1024)
IMIN = np.int32(-(2 ** 31))
IMAX = np.int32(2 ** 31 - 1)


def _mesh():
    return plsc.VectorSubcoreMesh(core_axis_name="c", subcore_axis_name="s")


def _full(v):
    return jnp.full((L,), v, jnp.int32)


def _key16(v_f32):
    b = lax.bitcast_convert_type(v_f32, jnp.int32)
    m = lax.shift_right_arithmetic(b, _full(31))
    return b ^ (m | IMIN)


def _merge_find(partv, nb, r_rem):
    iota = lax.iota(jnp.int32, L)

    def body(c, carry):
        bin_, below, total, ceq = carry
        acc = partv[pl.ds(c * L, L)] + partv[pl.ds(nb + c * L, L)]
        cs = plsc.cumsum(acc)
        mask = (total + cs) > r_rem
        found_here = jnp.any(mask)
        ffs = jnp.where(found_here, plsc.all_reduce_ffs(mask), np.int32(L))
        already = bin_ >= 0
        new_here = jnp.logical_and(found_here, jnp.logical_not(already))
        bin_ = jnp.where(new_here, c * L + ffs, bin_)
        below_here = jnp.sum(jnp.where(iota < ffs, acc, 0))
        eq_here = jnp.sum(jnp.where(iota == ffs, acc, 0))
        upd = jnp.logical_not(already)
        below = jnp.where(upd, below + below_here, below)
        ceq = jnp.where(jnp.logical_and(upd, found_here), eq_here, ceq)
        total = total + jnp.sum(acc)
        return bin_, below, total, ceq

    bin_, below, _, ceq = lax.fori_loop(
        0, nb // L, body, (_full(-1), _full(0), _full(0), _full(0)))
    return bin_, below, ceq


def _merge_find_h1(partv, r_rem):
    nb = NBS[0]
    iota = lax.iota(jnp.int32, L)

    def scan(acc, r_rem, carry, bin_of_ffs):
        bin_, below, total = carry
        cs = plsc.cumsum(acc)
        mask = (total + cs) > r_rem
        found = jnp.any(mask)
        ffs = jnp.where(found, plsc.all_reduce_ffs(mask), np.int32(L))
        already = bin_ >= 0
        new = jnp.logical_and(found, jnp.logical_not(already))
        bin_ = jnp.where(new, bin_of_ffs(ffs), bin_)
        below_here = jnp.sum(jnp.where(iota < ffs, acc, 0))
        below = jnp.where(already, below, below + below_here)
        total = total + jnp.sum(acc)
        return bin_, below, total

    def bodyA(g, carry):
        off = (nb - L) - g * L
        acc = lax.rev(partv[pl.ds(off, L)] + partv[pl.ds(nb + off, L)], (0,))
        return scan(acc, r_rem, carry, lambda f: (nb - 1) - g * L - f)

    def bodyB(g, carry):
        off = g * L
        acc = partv[pl.ds(off, L)] + partv[pl.ds(nb + off, L)]
        return scan(acc, r_rem, carry, lambda f: g * L + f)

    carry = (_full(-1), _full(0), _full(0))
    carry = lax.fori_loop(0, nb // (2 * L), bodyA, carry)
    bin_, below, _ = lax.fori_loop(0, nb // (2 * L), bodyB, carry)
    return bin_, below


def _step(h_hbm, partv, p, prefix, r_rem):
    nb = NBS[p]
    pltpu.sync_copy(h_hbm.at[pl.ds(0, 2 * nb)], partv.at[pl.ds(0, 2 * nb)])
    b_, below, ceq = _merge_find(partv, nb, r_rem)
    prefix = (prefix << int(np.log2(nb))) | b_
    return prefix, r_rem - below, ceq


def _make_pass(p):
    nb = NBS[p]
    shift = SHIFTS[p]
    out_words = 2 * nb + (W * L if p == 2 else 0) + (2 * L if p else 0)
    state_off = 2 * nb + (W * L if p == 2 else 0)
    scratch = [
        pltpu.VMEM((CHUNK,), jnp.float32),
        pltpu.VMEM((nb,), jnp.int32),
        pltpu.VMEM((nb,), jnp.int32),
        pltpu.VMEM_SHARED((nb,), jnp.int32),
        pltpu.SemaphoreType.DMA,
    ]
    if p:
        scratch.append(pltpu.VMEM((2 * NBS[p - 1],), jnp.int32))

    @functools.partial(
        pl.kernel,
        out_type=jax.ShapeDtypeStruct((out_words,), jnp.int32),
        mesh=_mesh(),
        compiler_params=pltpu.CompilerParams(needs_layout_passes=False),
        scratch_types=scratch,
    )
    def body(*args):
        x_hbm = args[0]
        if p:
            h_prev = args[1]
            out_hbm = args[2]
            xv, histv, idxv, shared, sem, partv = args[3:]
        else:
            out_hbm = args[1]
            xv, histv, idxv, shared, sem = args[2:]
            partv = None
        sid = lax.axis_index("s")
        core = lax.axis_index("c")
        wid = sid * NC + core
        cps = [
            pltpu.async_copy(
                x_hbm.at[wid * RPW + r], xv.at[pl.ds(r * COLS, COLS)], sem)
            for r in range(RPW)
        ]

        iota = lax.iota(jnp.int32, L)
        zero = jnp.zeros((L,), jnp.int32)

        @plsc.parallel_loop(0, nb // L, unroll=8)
        def _(c):
            histv[pl.ds(c * L, L)] = zero
            idxv[pl.ds(c * L, L)] = iota + c * L

        @pl.when(sid == 0)
        def _():
            pltpu.sync_copy(histv, shared)
        plsc.subcore_barrier()

        if p == 1:
            pltpu.sync_copy(h_prev.at[pl.ds(0, 2 * NBS[0])],
                            partv.at[pl.ds(0, 2 * NBS[0])])
            prefix, below = _merge_find_h1(partv, _full(RANK))
            r_rem = _full(RANK) - below
            sflip = jnp.where(prefix >= np.int32(NBS[0] // 2),
                              _full(NBS[1] - 1), _full(0))
        elif p == 2:
            poff = 2 * NBS[p - 1] + 2 * L
            pltpu.sync_copy(h_prev.at[pl.ds(poff - 2 * L, 2 * L)],
                            partv.at[pl.ds(0, 2 * L)])
            p1raw = partv[pl.ds(0, L)]
            prev_r = partv[pl.ds(L, L)]
            pltpu.sync_copy(h_prev.at[pl.ds(0, 2 * NBS[1])],
                            partv.at[pl.ds(0, 2 * NBS[1])])
            bin2, below, _ = _merge_find(partv, NBS[1], prev_r)
            r_rem = prev_r - below
            neg = p1raw >= np.int32(NBS[0] // 2)
            bin2_raw = bin2 ^ jnp.where(neg, _full(NBS[1] - 1), _full(0))
            raw22 = (p1raw << 11) | bin2_raw
            prefix = raw22 ^ jnp.where(neg, _full(0x3FFFFF), _full(0x200000))
        if p:
            @pl.when(jnp.logical_and(sid == 0, core == 0))
            def _():
                partv[pl.ds(0, L)] = prefix
                partv[pl.ds(L, L)] = r_rem
                pltpu.sync_copy(partv.at[pl.ds(0, 2 * L)],
                                out_hbm.at[pl.ds(state_off, 2 * L)])
        for cp in cps:
            cp.wait()

        ones = jnp.ones((L,), jnp.int32)
        shift_v = _full(shift)
        hi_v = _full(shift + int(np.log2(nb)))
        mask_dig = _full(nb - 1)

        @plsc.parallel_loop(0, VECS, carry=jnp.full((L,), IMAX, jnp.int32),
                            unroll=16)
        def mn(i, mn):
            if p == 0:
                b = lax.bitcast_convert_type(xv[pl.ds(i * L, L)], jnp.int32)
                dig = lax.shift_right_logical(b, shift_v)
                plsc.addupdate_scatter(histv, [dig], ones)
            elif p == 1:
                b = lax.bitcast_convert_type(xv[pl.ds(i * L, L)], jnp.int32)
                hi = lax.shift_right_logical(b, hi_v)
                dig = (lax.shift_right_logical(b, shift_v) & mask_dig) ^ sflip
                plsc.addupdate_scatter(histv, [dig], ones, mask=hi == prefix)
            else:
                key = _key16(xv[pl.ds(i * L, L)])
                dig = key & mask_dig
                hi = lax.shift_right_logical(key, hi_v)
                plsc.addupdate_scatter(histv, [dig], ones, mask=hi == prefix)
                mn = jnp.where(hi > prefix,
                               jnp.minimum(mn, key ^ IMIN), mn)
            return mn

        pltpu.sync_copy(histv, shared.at[idxv], add=True)
        plsc.subcore_barrier()

        @pl.when(sid == 0)
        def _():
            pltpu.sync_copy(shared, out_hbm.at[pl.ds(core * nb, nb)])

        if p == 2:
            idxv[pl.ds(0, L)] = mn
            pltpu.sync_copy(
                idxv.at[pl.ds(0, L)],
                out_hbm.at[pl.ds(2 * nb + wid * L, L)])

    return body


_pass1 = _make_pass(0)
_pass2 = _make_pass(1)
_pass3 = _make_pass(2)


@functools.partial(
    pl.kernel,
    out_type=jax.ShapeDtypeStruct((L,), jnp.float32),
    mesh=_mesh(),
    compiler_params=pltpu.CompilerParams(needs_layout_passes=False),
    scratch_types=[
        pltpu.VMEM((2 * NBS[2],), jnp.int32),
        pltpu.VMEM((W * L,), jnp.int32),
        pltpu.VMEM((L,), jnp.float32),
    ],
)
def _final(h3m, out_hbm, partv, cv, outv):
    sid = lax.axis_index("s")
    core = lax.axis_index("c")
    iota = lax.iota(jnp.int32, L)
    nb3 = NBS[2]

    @pl.when(jnp.logical_and(sid == 0, core == 0))
    def _():
        soff = 2 * nb3 + W * L
        pltpu.sync_copy(h3m.at[pl.ds(soff, 2 * L)], cv.at[pl.ds(0, 2 * L)])
        prev_prefix = cv[pl.ds(0, L)]
        prev_r = cv[pl.ds(L, L)]
        key_a, r_rem, ceq = _step(h3m, partv, 2, prev_prefix, prev_r)
        bin3 = key_a & (nb3 - 1)

        def nzbody(c, nxt):
            acc = partv[pl.ds(c * L, L)] + partv[pl.ds(nb3 + c * L, L)]
            gi = c * L + iota
            cand = jnp.where(jnp.logical_and(acc > 0, gi > bin3),
                             gi, _full(nb3))
            return jnp.minimum(nxt, cand)

        nxt = lax.fori_loop(0, nb3 // L, nzbody, _full(nb3))
        nxt_bin = jnp.min(nxt)
        keyc_cand = jnp.where(
            nxt_bin < nb3,
            (((key_a >> 10) << 10) | nxt_bin) ^ IMIN, IMAX)

        pltpu.sync_copy(h3m.at[pl.ds(2 * nb3, W * L)], cv)

        def mbody(t, m_acc):
            return jnp.minimum(m_acc, cv[pl.ds(t * L, L)])

        mn_next = jnp.min(lax.fori_loop(
            0, W, mbody, jnp.full((L,), IMAX, jnp.int32)))

        cnt_le = (_full(RANK) - r_rem) + ceq
        keyc_b = jnp.where(cnt_le >= np.int32(RANK + 2),
                           key_a ^ IMIN,
                           jnp.minimum(keyc_cand, mn_next))
        key_b = keyc_b ^ IMIN
        keys = jnp.where(iota == 1, key_b, key_a)
        bits = jnp.where(keys < 0, keys ^ IMIN, ~keys)
        outv[...] = lax.bitcast_convert_type(bits, jnp.float32)
        pltpu.sync_copy(outv, out_hbm)


def kernel(x):
    h1 = _pass1(x)
    h2 = _pass2(x, h1)
    h3 = _pass3(x, h2)
    out = _final(h3)
    return out[0], out[1], out[2]


# speedup vs baseline: 1.0092x; 1.0092x over previous
import functools

import jax
import numpy as np
import jax.numpy as jnp
from jax import lax
from jax.experimental import pallas as pl
from jax.experimental.pallas import tpu as pltpu
from jax.experimental.pallas import tpu_sc as plsc

ROWS, COLS = 64, 32768
N = ROWS * COLS
RANK = N // 2 - 1
L = 16
NC, NS = 2, 16
W = NC * NS
RPW = ROWS // W
CHUNK = N // W
VECS = CHUNK // L
SHIFTS = (21, 10, 0)
NBS = (2048, 2048, 1024)
IMIN = np.int32(-(2 ** 31))
IMAX = np.int32(2 ** 31 - 1)


def _mesh():
    return plsc.VectorSubcoreMesh(core_axis_name="c", subcore_axis_name="s")


def _full(v):
    return jnp.full((L,), v, jnp.int32)


def _key16(v_f32):
    b = lax.bitcast_convert_type(v_f32, jnp.int32)
    m = lax.shift_right_arithmetic(b, _full(31))
    return b ^ (m | IMIN)


def _merge_find(partv, nb, r_rem):
    iota = lax.iota(jnp.int32, L)

    def body(c, carry):
        bin_, below, total, ceq = carry
        acc = partv[pl.ds(c * L, L)] + partv[pl.ds(nb + c * L, L)]
        cs = plsc.cumsum(acc)
        mask = (total + cs) > r_rem
        found_here = jnp.any(mask)
        ffs = jnp.where(found_here, plsc.all_reduce_ffs(mask), np.int32(L))
        already = bin_ >= 0
        new_here = jnp.logical_and(found_here, jnp.logical_not(already))
        bin_ = jnp.where(new_here, c * L + ffs, bin_)
        below_here = jnp.sum(jnp.where(iota < ffs, acc, 0))
        eq_here = jnp.sum(jnp.where(iota == ffs, acc, 0))
        upd = jnp.logical_not(already)
        below = jnp.where(upd, below + below_here, below)
        ceq = jnp.where(jnp.logical_and(upd, found_here), eq_here, ceq)
        total = total + jnp.sum(acc)
        return bin_, below, total, ceq

    bin_, below, _, ceq = lax.fori_loop(
        0, nb // L, body, (_full(-1), _full(0), _full(0), _full(0)))
    return bin_, below, ceq


def _step(h_hbm, partv, p, prefix, r_rem):
    nb = NBS[p]
    pltpu.sync_copy(h_hbm.at[pl.ds(0, 2 * nb)], partv.at[pl.ds(0, 2 * nb)])
    b_, below, ceq = _merge_find(partv, nb, r_rem)
    prefix = (prefix << int(np.log2(nb))) | b_
    return prefix, r_rem - below, ceq


def _make_pass(p):
    nb = NBS[p]
    shift = SHIFTS[p]
    out_words = 2 * nb + (W * L if p == 2 else 0) + (2 * L if p else 0)
    state_off = 2 * nb + (W * L if p == 2 else 0)
    scratch = [
        pltpu.VMEM((CHUNK,), jnp.float32),
        pltpu.VMEM((nb,), jnp.int32),
        pltpu.VMEM((nb,), jnp.int32),
        pltpu.VMEM_SHARED((nb,), jnp.int32),
        pltpu.SemaphoreType.DMA,
    ]
    if p:
        scratch.append(pltpu.VMEM((2 * NBS[p - 1],), jnp.int32))

    @functools.partial(
        pl.kernel,
        out_type=jax.ShapeDtypeStruct((out_words,), jnp.int32),
        mesh=_mesh(),
        compiler_params=pltpu.CompilerParams(needs_layout_passes=False),
        scratch_types=scratch,
    )
    def body(*args):
        x_hbm = args[0]
        if p:
            h_prev = args[1]
            out_hbm = args[2]
            xv, histv, idxv, shared, sem, partv = args[3:]
        else:
            out_hbm = args[1]
            xv, histv, idxv, shared, sem = args[2:]
            partv = None
        sid = lax.axis_index("s")
        core = lax.axis_index("c")
        wid = sid * NC + core
        cps = [
            pltpu.async_copy(
                x_hbm.at[wid * RPW + r], xv.at[pl.ds(r * COLS, COLS)], sem)
            for r in range(RPW)
        ]

        iota = lax.iota(jnp.int32, L)
        zero = jnp.zeros((L,), jnp.int32)

        @plsc.parallel_loop(0, nb // L, unroll=8)
        def _(c):
            histv[pl.ds(c * L, L)] = zero
            idxv[pl.ds(c * L, L)] = iota + c * L

        @pl.when(sid == 0)
        def _():
            pltpu.sync_copy(histv, shared)
        plsc.subcore_barrier()

        if p == 1:
            prefix, r_rem, _ = _step(h_prev, partv, 0, _full(0), _full(RANK))
        elif p == 2:
            poff = 2 * NBS[p - 1]
            pltpu.sync_copy(h_prev.at[pl.ds(poff, 2 * L)],
                            partv.at[pl.ds(0, 2 * L)])
            prev_prefix = partv[pl.ds(0, L)]
            prev_r = partv[pl.ds(L, L)]
            prefix, r_rem, _ = _step(h_prev, partv, 1, prev_prefix, prev_r)
        if p:
            @pl.when(jnp.logical_and(sid == 0, core == 0))
            def _():
                partv[pl.ds(0, L)] = prefix
                partv[pl.ds(L, L)] = r_rem
                pltpu.sync_copy(partv.at[pl.ds(0, 2 * L)],
                                out_hbm.at[pl.ds(state_off, 2 * L)])
        for cp in cps:
            cp.wait()

        ones = jnp.ones((L,), jnp.int32)
        shift_v = _full(shift)
        hi_v = _full(shift + int(np.log2(nb)))
        mask_dig = _full(nb - 1)

        @plsc.parallel_loop(0, VECS, carry=jnp.full((L,), IMAX, jnp.int32),
                            unroll=16)
        def mn(i, mn):
            key = _key16(xv[pl.ds(i * L, L)])
            if shift:
                dig = lax.shift_right_logical(key, shift_v) & mask_dig
            else:
                dig = key & mask_dig
            if p:
                hi = lax.shift_right_logical(key, hi_v)
                m = hi == prefix
                plsc.addupdate_scatter(histv, [dig], ones, mask=m)
                if p == 2:
                    mn = jnp.where(hi > prefix,
                                   jnp.minimum(mn, key ^ IMIN), mn)
            else:
                plsc.addupdate_scatter(histv, [dig], ones)
            return mn

        pltpu.sync_copy(histv, shared.at[idxv], add=True)
        plsc.subcore_barrier()

        @pl.when(sid == 0)
        def _():
            pltpu.sync_copy(shared, out_hbm.at[pl.ds(core * nb, nb)])

        if p == 2:
            idxv[pl.ds(0, L)] = mn
            pltpu.sync_copy(
                idxv.at[pl.ds(0, L)],
                out_hbm.at[pl.ds(2 * nb + wid * L, L)])

    return body


_pass1 = _make_pass(0)
_pass2 = _make_pass(1)
_pass3 = _make_pass(2)


@functools.partial(
    pl.kernel,
    out_type=jax.ShapeDtypeStruct((L,), jnp.float32),
    mesh=_mesh(),
    compiler_params=pltpu.CompilerParams(needs_layout_passes=False),
    scratch_types=[
        pltpu.VMEM((2 * NBS[2],), jnp.int32),
        pltpu.VMEM((W * L,), jnp.int32),
        pltpu.VMEM((L,), jnp.float32),
    ],
)
def _final(h3m, out_hbm, partv, cv, outv):
    sid = lax.axis_index("s")
    core = lax.axis_index("c")
    iota = lax.iota(jnp.int32, L)
    nb3 = NBS[2]

    @pl.when(jnp.logical_and(sid == 0, core == 0))
    def _():
        soff = 2 * nb3 + W * L
        pltpu.sync_copy(h3m.at[pl.ds(soff, 2 * L)], cv.at[pl.ds(0, 2 * L)])
        prev_prefix = cv[pl.ds(0, L)]
        prev_r = cv[pl.ds(L, L)]
        key_a, r_rem, ceq = _step(h3m, partv, 2, prev_prefix, prev_r)
        bin3 = key_a & (nb3 - 1)

        def nzbody(c, nxt):
            acc = partv[pl.ds(c * L, L)] + partv[pl.ds(nb3 + c * L, L)]
            gi = c * L + iota
            cand = jnp.where(jnp.logical_and(acc > 0, gi > bin3),
                             gi, _full(nb3))
            return jnp.minimum(nxt, cand)

        nxt = lax.fori_loop(0, nb3 // L, nzbody, _full(nb3))
        nxt_bin = jnp.min(nxt)
        keyc_cand = jnp.where(
            nxt_bin < nb3,
            (((key_a >> 10) << 10) | nxt_bin) ^ IMIN, IMAX)

        pltpu.sync_copy(h3m.at[pl.ds(2 * nb3, W * L)], cv)

        def mbody(t, m_acc):
            return jnp.minimum(m_acc, cv[pl.ds(t * L, L)])

        mn_next = jnp.min(lax.fori_loop(
            0, W, mbody, jnp.full((L,), IMAX, jnp.int32)))

        cnt_le = (_full(RANK) - r_rem) + ceq
        keyc_b = jnp.where(cnt_le >= np.int32(RANK + 2),
                           key_a ^ IMIN,
                           jnp.minimum(keyc_cand, mn_next))
        key_b = keyc_b ^ IMIN
        keys = jnp.where(iota == 1, key_b, key_a)
        bits = jnp.where(keys < 0, keys ^ IMIN, ~keys)
        outv[...] = lax.bitcast_convert_type(bits, jnp.float32)
        pltpu.sync_copy(outv, out_hbm)


def kernel(x):
    h1 = _pass1(x)
    h2 = _pass2(x, h1)
    h3 = _pass3(x, h2)
    out = _final(h3)
    return out[0], out[1], out[2]
